# initial kernel scaffold (unmeasured)
import jax
import jax.numpy as jnp
from jax import lax
from jax.experimental import pallas as pl
from jax.experimental.pallas import tpu as pltpu

F_TILE = 512


def kernel(x, dy):
    k, d = x.shape
    k2, f = dy.shape
    assert k == k2
    out_rows = d // 2
    n_steps = f // F_TILE

    def body(x_ref, dy_ref, out_ref, xbf_ref, send_ref, recv_ref,
             send_sems, recv_sems):
        j = pl.program_id(0)
        my_x = lax.axis_index("x")
        my_y = lax.axis_index("y")
        my_z = lax.axis_index("z")
        slot = lax.rem(j, 2)

        @pl.when(j == 0)
        def _():
            xbf_ref[...] = x_ref[...].astype(jnp.bfloat16)

        dybf = dy_ref[...].astype(jnp.bfloat16)
        res = lax.dot_general(
            xbf_ref[...], dybf,
            dimension_numbers=(((0,), (0,)), ((), ())),
            preferred_element_type=jnp.float32,
        )
        mine = lax.dynamic_slice_in_dim(res, my_x * out_rows, out_rows, 0)
        theirs = lax.dynamic_slice_in_dim(res, (1 - my_x) * out_rows, out_rows, 0)
        send_ref[slot] = theirs.astype(jnp.bfloat16)

        rdma = pltpu.make_async_remote_copy(
            src_ref=send_ref.at[slot],
            dst_ref=recv_ref.at[slot],
            send_sem=send_sems.at[slot],
            recv_sem=recv_sems.at[slot],
            device_id=(1 - my_x, my_y, my_z),
            device_id_type=pl.DeviceIdType.MESH,
        )
        rdma.start()
        rdma.wait()

        out_ref[...] = mine + recv_ref[slot].astype(jnp.float32)

    return pl.pallas_call(
        body,
        grid=(n_steps,),
        out_shape=jax.ShapeDtypeStruct((out_rows, f), jnp.float32),
        in_specs=[
            pl.BlockSpec((k, d), lambda j: (0, 0)),
            pl.BlockSpec((k, F_TILE), lambda j: (0, j)),
        ],
        out_specs=pl.BlockSpec((out_rows, F_TILE), lambda j: (0, j)),
        scratch_shapes=[
            pltpu.VMEM((k, d), jnp.bfloat16),
            pltpu.VMEM((2, out_rows, F_TILE), jnp.bfloat16),
            pltpu.VMEM((2, out_rows, F_TILE), jnp.bfloat16),
            pltpu.SemaphoreType.DMA((2,)),
            pltpu.SemaphoreType.DMA((2,)),
        ],
        compiler_params=pltpu.CompilerParams(
            dimension_semantics=("arbitrary",),
            collective_id=0,
        ),
    )(x, dy)


# baseline (device time: 333123 ns/iter reference)
import jax
import jax.numpy as jnp
from jax import lax
from jax.experimental import pallas as pl
from jax.experimental.pallas import tpu as pltpu

F_TILE = 512


def kernel(x, dy):
    k, d = x.shape
    k2, f = dy.shape
    assert k == k2
    out_rows = d // 2
    n_steps = f // F_TILE

    def body(x_ref, dy_ref, out_ref, xbf_ref, send_ref, recv_ref,
             send_sems, recv_sems):
        j = pl.program_id(0)
        my_x = lax.axis_index("x")
        my_y = lax.axis_index("y")
        my_z = lax.axis_index("z")
        slot = lax.rem(j, 2)

        @pl.when(j == 0)
        def _():
            xbf_ref[...] = x_ref[...].astype(jnp.bfloat16)

        dybf = dy_ref[...].astype(jnp.bfloat16)
        res = lax.dot_general(
            xbf_ref[...], dybf,
            dimension_numbers=(((0,), (0,)), ((), ())),
            preferred_element_type=jnp.float32,
        )
        top = res[:out_rows, :]
        bot = res[out_rows:, :]
        is0 = my_x == 0
        mine = jnp.where(is0, top, bot)
        theirs = jnp.where(is0, bot, top)
        send_ref[slot] = theirs.astype(jnp.bfloat16)

        rdma = pltpu.make_async_remote_copy(
            src_ref=send_ref.at[slot],
            dst_ref=recv_ref.at[slot],
            send_sem=send_sems.at[slot],
            recv_sem=recv_sems.at[slot],
            device_id=(1 - my_x, my_y, my_z),
            device_id_type=pl.DeviceIdType.MESH,
        )
        rdma.start()
        rdma.wait()

        out_ref[...] = mine + recv_ref[slot].astype(jnp.float32)

    return pl.pallas_call(
        body,
        grid=(n_steps,),
        out_shape=jax.ShapeDtypeStruct((out_rows, f), jnp.float32),
        in_specs=[
            pl.BlockSpec((k, d), lambda j: (0, 0)),
            pl.BlockSpec((k, F_TILE), lambda j: (0, j)),
        ],
        out_specs=pl.BlockSpec((out_rows, F_TILE), lambda j: (0, j)),
        scratch_shapes=[
            pltpu.VMEM((k, d), jnp.bfloat16),
            pltpu.VMEM((2, out_rows, F_TILE), jnp.bfloat16),
            pltpu.VMEM((2, out_rows, F_TILE), jnp.bfloat16),
            pltpu.SemaphoreType.DMA((2,)),
            pltpu.SemaphoreType.DMA((2,)),
        ],
        compiler_params=pltpu.CompilerParams(
            dimension_semantics=("arbitrary",),
        ),
    )(x, dy)


# device time: 144453 ns/iter; 2.3061x vs baseline; 2.3061x over previous
import jax
import jax.numpy as jnp
from jax import lax
from jax.experimental import pallas as pl
from jax.experimental.pallas import tpu as pltpu

F_TILE = 512
NSLOT = 8
NMINE = 4


def kernel(x, dy):
    k, d = x.shape
    k2, f = dy.shape
    assert k == k2
    half = d // 2
    qr = half // 4
    nt = f // F_TILE
    grid = nt + 3

    def body(x_ref, dy_ref, out_ref, xbf_ref, mine_ref,
             xsend_ref, xrecv_ref, zrecv_ref, yrecv_ref,
             xs_s, xs_r, zs_s, zs_r, ya_s, ya_r, yb_s, yb_r):
        j = pl.program_id(0)
        my_x = lax.axis_index("x")
        my_y = lax.axis_index("y")
        my_z = lax.axis_index("z")
        x_nbr = (1 - my_x, my_y, my_z)
        z_nbr = (my_x, my_y, 1 - my_z)
        y_nbr = (my_x, 1 - my_y, my_z)

        def x_rdma(s):
            return pltpu.make_async_remote_copy(
                src_ref=xsend_ref.at[s], dst_ref=xrecv_ref.at[s],
                send_sem=xs_s.at[s], recv_sem=xs_r.at[s],
                device_id=x_nbr, device_id_type=pl.DeviceIdType.MESH)

        def z_rdma(s):
            return pltpu.make_async_remote_copy(
                src_ref=xrecv_ref.at[s], dst_ref=zrecv_ref.at[s],
                send_sem=zs_s.at[s], recv_sem=zs_r.at[s],
                device_id=z_nbr, device_id_type=pl.DeviceIdType.MESH)

        def ya_rdma(s):
            return pltpu.make_async_remote_copy(
                src_ref=xrecv_ref.at[s], dst_ref=yrecv_ref.at[s, 0],
                send_sem=ya_s.at[s], recv_sem=ya_r.at[s],
                device_id=y_nbr, device_id_type=pl.DeviceIdType.MESH)

        def yb_rdma(s):
            return pltpu.make_async_remote_copy(
                src_ref=zrecv_ref.at[s], dst_ref=yrecv_ref.at[s, 1],
                send_sem=yb_s.at[s], recv_sem=yb_r.at[s],
                device_id=y_nbr, device_id_type=pl.DeviceIdType.MESH)

        @pl.when(j == 0)
        def _():
            q = 2 * my_y + my_z
            own = x_ref[:, pl.ds(my_x * half, half)]
            qcols = x_ref[:, pl.ds((1 - my_x) * half + q * qr, qr)]
            xbf_ref[:, :half] = own.astype(jnp.bfloat16)
            xbf_ref[:, half:] = qcols.astype(jnp.bfloat16)

        @pl.when(j < nt)
        def _():
            s0 = lax.rem(j, NSLOT)
            m0 = lax.rem(j, NMINE)
            dybf = dy_ref[...].astype(jnp.bfloat16)
            res = lax.dot_general(
                xbf_ref[...], dybf,
                dimension_numbers=(((0,), (0,)), ((), ())),
                preferred_element_type=jnp.float32,
            )
            mine_ref[m0] = res[:half, :].astype(jnp.bfloat16)
            xsend_ref[s0] = res[half:, :].astype(jnp.bfloat16)
            x_rdma(s0).start()

        @pl.when((j >= 1) & (j <= nt))
        def _():
            s1 = lax.rem(j + NSLOT - 1, NSLOT)
            r = x_rdma(s1)
            r.wait_send()
            r.wait_recv()
            z_rdma(s1).start()

        @pl.when((j >= 2) & (j <= nt + 1))
        def _():
            s2 = lax.rem(j + NSLOT - 2, NSLOT)
            r = z_rdma(s2)
            r.wait_send()
            r.wait_recv()
            ya_rdma(s2).start()
            yb_rdma(s2).start()

        @pl.when(j >= 3)
        def _():
            s3 = lax.rem(j + NSLOT - 3, NSLOT)
            m3 = lax.rem(j + NMINE - 3, NMINE)
            ra = ya_rdma(s3)
            rb = yb_rdma(s3)
            ra.wait_send()
            ra.wait_recv()
            rb.wait_send()
            rb.wait_recv()
            posx = 2 * my_y + my_z
            posz = 2 * my_y + (1 - my_z)
            posy0 = 2 * (1 - my_y) + my_z
            qx = xrecv_ref[s3].astype(jnp.float32)
            qz = zrecv_ref[s3].astype(jnp.float32)
            qy0 = yrecv_ref[s3, 0].astype(jnp.float32)
            qy1 = yrecv_ref[s3, 1].astype(jnp.float32)
            mine = mine_ref[m3].astype(jnp.float32)
            for p in range(4):
                rp = jnp.where(
                    posx == p, qx,
                    jnp.where(posz == p, qz,
                              jnp.where(posy0 == p, qy0, qy1)))
                out_ref[p * qr:(p + 1) * qr, :] = (
                    mine[p * qr:(p + 1) * qr, :] + rp)

    return pl.pallas_call(
        body,
        grid=(grid,),
        out_shape=jax.ShapeDtypeStruct((half, f), jnp.float32),
        in_specs=[
            pl.BlockSpec((k, d), lambda j: (0, 0)),
            pl.BlockSpec((k, F_TILE), lambda j: (0, jnp.minimum(j, nt - 1))),
        ],
        out_specs=pl.BlockSpec(
            (half, F_TILE), lambda j: (0, jnp.maximum(j - 3, 0))),
        scratch_shapes=[
            pltpu.VMEM((k, half + qr), jnp.bfloat16),
            pltpu.VMEM((NMINE, half, F_TILE), jnp.bfloat16),
            pltpu.VMEM((NSLOT, qr, F_TILE), jnp.bfloat16),
            pltpu.VMEM((NSLOT, qr, F_TILE), jnp.bfloat16),
            pltpu.VMEM((NSLOT, qr, F_TILE), jnp.bfloat16),
            pltpu.VMEM((NSLOT, 2, qr, F_TILE), jnp.bfloat16),
            pltpu.SemaphoreType.DMA((NSLOT,)),
            pltpu.SemaphoreType.DMA((NSLOT,)),
            pltpu.SemaphoreType.DMA((NSLOT,)),
            pltpu.SemaphoreType.DMA((NSLOT,)),
            pltpu.SemaphoreType.DMA((NSLOT,)),
            pltpu.SemaphoreType.DMA((NSLOT,)),
            pltpu.SemaphoreType.DMA((NSLOT,)),
            pltpu.SemaphoreType.DMA((NSLOT,)),
        ],
        compiler_params=pltpu.CompilerParams(
            dimension_semantics=("arbitrary",),
            vmem_limit_bytes=100 * 1024 * 1024,
        ),
    )(x, dy)


# device time: 125646 ns/iter; 2.6513x vs baseline; 1.1497x over previous
import jax
import jax.numpy as jnp
from jax import lax
from jax.experimental import pallas as pl
from jax.experimental.pallas import tpu as pltpu

F_TILE = 512
FT2 = F_TILE // 2
NSLOT = 8
NMINE = 4

SEM_X, SEM_H1Z, SEM_H1Y, SEM_H2Y0, SEM_H2Y1, SEM_H2Z0, SEM_H2Z1 = range(7)


def kernel(x, dy):
    k, d = x.shape
    k2, f = dy.shape
    assert k == k2
    half = d // 2
    qr = half // 4
    nt = f // F_TILE
    grid = nt + 3

    def body(x_ref, dy_ref, out_ref, xbf_ref, mine_ref,
             xsend_ref, xrecv_ref, h1z_ref, h1y_ref, h2y_ref, h2z_ref,
             send_sems, recv_sems):
        j = pl.program_id(0)
        my_x = lax.axis_index("x")
        my_y = lax.axis_index("y")
        my_z = lax.axis_index("z")
        x_nbr = (1 - my_x, my_y, my_z)
        z_nbr = (my_x, my_y, 1 - my_z)
        y_nbr = (my_x, 1 - my_y, my_z)

        def rdma(kind, s, src, dst, nbr):
            return pltpu.make_async_remote_copy(
                src_ref=src, dst_ref=dst,
                send_sem=send_sems.at[kind, s], recv_sem=recv_sems.at[kind, s],
                device_id=nbr, device_id_type=pl.DeviceIdType.MESH)

        def x_rdma(s):
            return rdma(SEM_X, s, xsend_ref.at[s], xrecv_ref.at[s], x_nbr)

        def h1z_rdma(s):
            return rdma(SEM_H1Z, s, xrecv_ref.at[s, 0], h1z_ref.at[s], z_nbr)

        def h1y_rdma(s):
            return rdma(SEM_H1Y, s, xrecv_ref.at[s, 1], h1y_ref.at[s], y_nbr)

        def h2y0_rdma(s):
            return rdma(SEM_H2Y0, s, xrecv_ref.at[s, 0], h2y_ref.at[s, 0], y_nbr)

        def h2y1_rdma(s):
            return rdma(SEM_H2Y1, s, h1z_ref.at[s], h2y_ref.at[s, 1], y_nbr)

        def h2z0_rdma(s):
            return rdma(SEM_H2Z0, s, xrecv_ref.at[s, 1], h2z_ref.at[s, 0], z_nbr)

        def h2z1_rdma(s):
            return rdma(SEM_H2Z1, s, h1y_ref.at[s], h2z_ref.at[s, 1], z_nbr)

        @pl.when(j == 0)
        def _():
            q = 2 * my_y + my_z
            own = x_ref[:, pl.ds(my_x * half, half)]
            qcols = x_ref[:, pl.ds((1 - my_x) * half + q * qr, qr)]
            xbf_ref[:, :half] = own.astype(jnp.bfloat16)
            xbf_ref[:, half:] = qcols.astype(jnp.bfloat16)

        @pl.when(j < nt)
        def _():
            s0 = lax.rem(j, NSLOT)
            m0 = lax.rem(j, NMINE)
            dybf = dy_ref[...].astype(jnp.bfloat16)
            res = lax.dot_general(
                xbf_ref[...], dybf,
                dimension_numbers=(((0,), (0,)), ((), ())),
                preferred_element_type=jnp.float32,
            )
            mine_ref[m0] = res[:half, :].astype(jnp.bfloat16)
            xsend_ref[s0, 0] = res[half:, :FT2].astype(jnp.bfloat16)
            xsend_ref[s0, 1] = res[half:, FT2:].astype(jnp.bfloat16)
            x_rdma(s0).start()

        @pl.when((j >= 1) & (j <= nt))
        def _():
            s1 = lax.rem(j + NSLOT - 1, NSLOT)
            r = x_rdma(s1)
            r.wait_send()
            r.wait_recv()
            h1z_rdma(s1).start()
            h1y_rdma(s1).start()

        @pl.when((j >= 2) & (j <= nt + 1))
        def _():
            s2 = lax.rem(j + NSLOT - 2, NSLOT)
            for mk in (h1z_rdma(s2), h1y_rdma(s2)):
                mk.wait_send()
                mk.wait_recv()
            h2y0_rdma(s2).start()
            h2y1_rdma(s2).start()
            h2z0_rdma(s2).start()
            h2z1_rdma(s2).start()

        @pl.when(j >= 3)
        def _():
            s3 = lax.rem(j + NSLOT - 3, NSLOT)
            m3 = lax.rem(j + NMINE - 3, NMINE)
            for mk in (h2y0_rdma(s3), h2y1_rdma(s3),
                       h2z0_rdma(s3), h2z1_rdma(s3)):
                mk.wait_send()
                mk.wait_recv()
            pieces = (
                (xrecv_ref.at[s3, 0], 2 * my_y + my_z, True),
                (h1z_ref.at[s3], 2 * my_y + (1 - my_z), True),
                (h2y_ref.at[s3, 0], 2 * (1 - my_y) + my_z, True),
                (h2y_ref.at[s3, 1], 2 * (1 - my_y) + (1 - my_z), True),
                (xrecv_ref.at[s3, 1], 2 * my_y + my_z, False),
                (h1y_ref.at[s3], 2 * (1 - my_y) + my_z, False),
                (h2z_ref.at[s3, 0], 2 * my_y + (1 - my_z), False),
                (h2z_ref.at[s3, 1], 2 * (1 - my_y) + (1 - my_z), False),
            )
            for piece, pos, is_a in pieces:
                rows = pl.ds(pos * qr, qr)
                cols = slice(0, FT2) if is_a else slice(FT2, F_TILE)
                out_ref[rows, cols] = (
                    mine_ref[m3, rows, cols].astype(jnp.float32)
                    + piece[...].astype(jnp.float32))

    return pl.pallas_call(
        body,
        grid=(grid,),
        out_shape=jax.ShapeDtypeStruct((half, f), jnp.float32),
        in_specs=[
            pl.BlockSpec((k, d), lambda j: (0, 0)),
            pl.BlockSpec((k, F_TILE), lambda j: (0, jnp.minimum(j, nt - 1))),
        ],
        out_specs=pl.BlockSpec(
            (half, F_TILE), lambda j: (0, jnp.maximum(j - 3, 0))),
        scratch_shapes=[
            pltpu.VMEM((k, half + qr), jnp.bfloat16),
            pltpu.VMEM((NMINE, half, F_TILE), jnp.bfloat16),
            pltpu.VMEM((NSLOT, 2, qr, FT2), jnp.bfloat16),
            pltpu.VMEM((NSLOT, 2, qr, FT2), jnp.bfloat16),
            pltpu.VMEM((NSLOT, qr, FT2), jnp.bfloat16),
            pltpu.VMEM((NSLOT, qr, FT2), jnp.bfloat16),
            pltpu.VMEM((NSLOT, 2, qr, FT2), jnp.bfloat16),
            pltpu.VMEM((NSLOT, 2, qr, FT2), jnp.bfloat16),
            pltpu.SemaphoreType.DMA((7, NSLOT)),
            pltpu.SemaphoreType.DMA((7, NSLOT)),
        ],
        compiler_params=pltpu.CompilerParams(
            dimension_semantics=("arbitrary",),
            vmem_limit_bytes=100 * 1024 * 1024,
        ),
    )(x, dy)
